# baseline (speedup 1.0000x reference)
import functools

import jax
import jax.numpy as jnp
import numpy as np
from jax import lax
from jax.experimental import pallas as pl
from jax.experimental.pallas import tpu as pltpu
from jax.experimental.pallas import tpu_sc as plsc

N = 10000
E = 320000
NC = 2
NS = 16
NW = NC * NS
EPW = E // NW
CH = 100
NCHUNK = EPW // CH
RPT = N // NS
ZR = 125


def _leaky(x):
    return jnp.where(x >= 0, x, 0.2 * x)


def _make_sc_edge(hw, rw, nb, ao, l2):
    mesh = plsc.VectorSubcoreMesh(core_axis_name="c", subcore_axis_name="s",
                                  num_cores=NC, num_subcores=NS)

    @functools.partial(
        pl.kernel,
        out_type=jax.ShapeDtypeStruct((NC, N, rw), jnp.float32),
        mesh=mesh,
        compiler_params=pltpu.CompilerParams(use_tc_tiling_on_sc=False),
        scratch_types=[
            pltpu.VMEM((NCHUNK, CH), jnp.int32),
            pltpu.VMEM((NCHUNK, CH), jnp.int32),
            pltpu.VMEM((2, CH, hw), jnp.float32),
            pltpu.VMEM((2, CH, 16), jnp.float32),
            pltpu.VMEM((2, CH, rw), jnp.float32),
            pltpu.VMEM((16,), jnp.float32),
            pltpu.VMEM((ZR, rw), jnp.float32),
            pltpu.VMEM_SHARED((N, rw), jnp.float32),
            pltpu.SemaphoreType.DMA,
            pltpu.SemaphoreType.DMA,
        ],
    )
    def sc_kernel(src_hbm, dst_hbm, ha_hbm, adst_hbm, bound_hbm,
                  acc_hbm, src_v, dst_v, ha_rows, ad_rows, res,
                  bnd_v, zb, acc_sh, sem0, sem1):
        c = lax.axis_index("c")
        s = lax.axis_index("s")
        wid = s * NC + c
        sems = (sem0, sem1)

        def zrow(r, _):
            for k in range(rw // 16):
                zb[r, pl.ds(16 * k, 16)] = jnp.zeros((16,), jnp.float32)
            return 0
        lax.fori_loop(0, ZR, zrow, 0)
        for j in range(RPT // ZR):
            pltpu.sync_copy(zb, acc_sh.at[pl.ds(s * RPT + j * ZR, ZR)])
        plsc.subcore_barrier()

        pltpu.sync_copy(bound_hbm, bnd_v)
        pltpu.sync_copy(src_hbm.at[pl.ds(wid * NCHUNK, NCHUNK)], src_v)
        pltpu.sync_copy(dst_hbm.at[pl.ds(wid * NCHUNK, NCHUNK)], dst_v)
        bnd = bnd_v[...]
        lane = lax.iota(jnp.int32, 16)

        def gathers(i, b):
            return (pltpu.make_async_copy(ha_hbm.at[src_v.at[i]],
                                          ha_rows.at[b], sems[b]),
                    pltpu.make_async_copy(adst_hbm.at[dst_v.at[i]],
                                          ad_rows.at[b], sems[b]))

        def compute(i, b):
            @plsc.parallel_loop(0, CH, step=1, unroll=8)
            def _edge(e):
                ex = jnp.exp(_leaky(ha_rows[b, e, pl.ds(ao, 16)]
                                    + ad_rows[b, e, :]) - bnd)
                for k in range(nb):
                    rb = ha_rows[b, e, pl.ds(16 * k, 16)] * ex
                    if l2 and k == nb - 1:
                        rb = jnp.where(lane < 8, rb,
                                       jnp.where(lane == 8, ex,
                                                 jnp.zeros((16,), jnp.float32)))
                    res[b, e, pl.ds(16 * k, 16)] = rb
                if not l2:
                    res[b, e, pl.ds(16 * nb, 16)] = ex
            pltpu.sync_copy(res.at[b], acc_sh.at[dst_v.at[i]], add=True)

        for d in gathers(0, 0):
            d.start()

        def dbl(g, _):
            i0 = 2 * g
            d1 = gathers(i0 + 1, 1)
            for d in d1:
                d.start()
            for d in gathers(i0, 0):
                d.wait()
            compute(i0, 0)
            for d in gathers(i0 + 2, 0):
                d.start()
            for d in d1:
                d.wait()
            compute(i0 + 1, 1)
            return 0
        if NCHUNK % 2:
            lax.fori_loop(0, NCHUNK // 2, dbl, 0)
            for d in gathers(NCHUNK - 1, 0):
                d.wait()
            compute(NCHUNK - 1, 0)
        else:
            lax.fori_loop(0, NCHUNK // 2 - 1, dbl, 0)
            dlast = gathers(NCHUNK - 1, 1)
            for d in dlast:
                d.start()
            for d in gathers(NCHUNK - 2, 0):
                d.wait()
            compute(NCHUNK - 2, 0)
            for d in dlast:
                d.wait()
            compute(NCHUNK - 1, 1)
        plsc.subcore_barrier()

        pltpu.sync_copy(acc_sh.at[pl.ds(s * RPT, RPT)],
                        acc_hbm.at[c, pl.ds(s * RPT, RPT)])

    return sc_kernel


_sc_cache = {}


def _sc_edge(hw, rw, nb, ao, l2):
    key = (hw, rw, nb, ao, l2)
    if key not in _sc_cache:
        _sc_cache[key] = _make_sc_edge(hw, rw, nb, ao, l2)
    return _sc_cache[key]


BLK = 2000
G = N // BLK


def _perm_mat(transpose=False):
    i = lax.broadcasted_iota(jnp.int32, (64, 64), 1 if transpose else 0)
    k = lax.broadcasted_iota(jnp.int32, (64, 64), 0 if transpose else 1)
    return (i == (k % 8) * 8 + k // 8).astype(jnp.float32)


def _sum_mat():
    j = lax.broadcasted_iota(jnp.int32, (64, 8), 0)
    h = lax.broadcasted_iota(jnp.int32, (64, 8), 1)
    return (j // 8 == h).astype(jnp.float32)


def _prep1_body(x_ref, w1_ref, as1_ref, ad1_ref, ha_ref, adst_ref, bnd_ref):
    i = pl.program_id(0)
    hm = jnp.dot(x_ref[...], w1_ref[...], preferred_element_type=jnp.float32)
    sm = _sum_mat()
    asrc = jnp.dot(hm * as1_ref[...], sm, preferred_element_type=jnp.float32)
    adst = jnp.dot(hm * ad1_ref[...], sm, preferred_element_type=jnp.float32)
    h = jnp.dot(hm, _perm_mat(), preferred_element_type=jnp.float32)
    ha_ref[...] = jnp.concatenate([h, asrc, asrc], axis=1)
    adst_ref[...] = jnp.concatenate([adst, adst], axis=1)
    cur = jnp.concatenate([jnp.max(asrc, axis=0),
                           jnp.max(adst, axis=0)]).reshape(1, 16)
    acc = jnp.where(i == 0, cur, jnp.maximum(bnd_ref[...], cur))
    bnd_ref[...] = acc

    @pl.when(i == G - 1)
    def _():
        b = jnp.maximum(bnd_ref[0, :8] + bnd_ref[0, 8:], 0.0)
        bnd_ref[...] = jnp.concatenate([b, b]).reshape(1, 16)


def _comb1_body(acc_ref, ha_ref, adst_ref, bnd_ref, b1_ref, g1_ref,
                be1_ref, w2_ref, as2_ref, ad2_ref,
                ha2_ref, adst2_ref, bnd2_ref):
    asrc = ha_ref[:, 64:72]
    adst = adst_ref[:, :8]
    exs = jnp.exp(_leaky(asrc + adst) - bnd_ref[0, :8][None, :])
    den = acc_ref[0, :, 64:72] + acc_ref[1, :, 64:72] + exs
    exs_t = jnp.concatenate([exs] * 8, axis=1)
    den_t = jnp.concatenate([den] * 8, axis=1)
    num = acc_ref[0, :, :64] + acc_ref[1, :, :64] + exs_t * ha_ref[:, :64]
    pm = _perm_mat()
    z = num / den_t + jnp.dot(b1_ref[...], pm)
    mu = jnp.mean(z, axis=-1, keepdims=True)
    var = jnp.mean((z - mu) ** 2, axis=-1, keepdims=True)
    z = (z - mu) * lax.rsqrt(var + 1e-5) * jnp.dot(g1_ref[...], pm) \
        + jnp.dot(be1_ref[...], pm)
    z = jnp.where(z > 0, z, jnp.exp(jnp.minimum(z, 0.0)) - 1.0)
    w2p = jnp.dot(_perm_mat(transpose=True), w2_ref[...],
                  preferred_element_type=jnp.float32)
    h2 = jnp.dot(z, w2p, preferred_element_type=jnp.float32)
    a2s = jnp.dot(h2, as2_ref[...], preferred_element_type=jnp.float32)
    a2d = jnp.dot(h2, ad2_ref[...], preferred_element_type=jnp.float32)
    ha2_ref[...] = jnp.concatenate(
        [h2] + [a2s] * 16 + [jnp.zeros((h2.shape[0], 8), jnp.float32)], axis=1)
    adst2_ref[...] = jnp.concatenate([a2d] * 16, axis=1)
    i = pl.program_id(0)
    cur = jnp.concatenate(
        [jnp.max(a2s).reshape(1), jnp.max(a2d).reshape(1),
         jnp.full((14,), -3.0e38, jnp.float32)]).reshape(1, 16)
    acc = jnp.where(i == 0, cur, jnp.maximum(bnd2_ref[...], cur))
    bnd2_ref[...] = acc

    @pl.when(i == G - 1)
    def _():
        b2 = jnp.maximum(bnd2_ref[0, 0] + bnd2_ref[0, 1], 0.0)
        bnd2_ref[...] = jnp.full((1, 16), b2, jnp.float32)


def _comb2_body(acc_ref, ha2_ref, adst2_ref, bnd2_ref, b2_ref,
                g2_ref, be2_ref, out_ref):
    ex2 = jnp.exp(_leaky(ha2_ref[:, 40:41] + adst2_ref[:, :1]) - bnd2_ref[0, 0])
    den = acc_ref[0, :, 40:41] + acc_ref[1, :, 40:41] + ex2
    num = acc_ref[0, :, :40] + acc_ref[1, :, :40] + ex2 * ha2_ref[:, :40]
    z = num / den + b2_ref[...][None, :]
    mu = jnp.mean(z, axis=-1, keepdims=True)
    var = jnp.mean((z - mu) ** 2, axis=-1, keepdims=True)
    out_ref[...] = (z - mu) * lax.rsqrt(var + 1e-5) * g2_ref[...][None, :] \
        + be2_ref[...][None, :]


def _row_spec(w):
    return pl.BlockSpec((BLK, w), lambda i: (i, 0))


def _full_spec(shape):
    nd = len(shape)
    return pl.BlockSpec(shape, lambda i: (0,) * nd)


def _acc_spec(w):
    return pl.BlockSpec((NC, BLK, w), lambda i: (0, i, 0))


def kernel(x, edge_index, W1, att_src1, att_dst1, bias1, gamma1, beta1,
           W2, att_src2, att_dst2, bias2, gamma2, beta2):
    src = edge_index[0].reshape(E // CH, CH)
    dst = edge_index[1].reshape(E // CH, CH)
    as2m = att_src2.reshape(40, 1)
    ad2m = att_dst2.reshape(40, 1)

    ha1, adst_t, bnd16 = pl.pallas_call(
        _prep1_body,
        grid=(G,),
        in_specs=[_row_spec(128), _full_spec((128, 64)), _full_spec((1, 64)),
                  _full_spec((1, 64))],
        out_specs=[_row_spec(80), _row_spec(16), _full_spec((1, 16))],
        out_shape=(jax.ShapeDtypeStruct((N, 80), jnp.float32),
                   jax.ShapeDtypeStruct((N, 16), jnp.float32),
                   jax.ShapeDtypeStruct((1, 16), jnp.float32)),
    )(x, W1, att_src1.reshape(1, 64), att_dst1.reshape(1, 64))

    acc1 = _sc_edge(80, 80, 4, 64, False)(src, dst, ha1, adst_t,
                                          bnd16.reshape(16))

    ha2, adst2_t, bnd2 = pl.pallas_call(
        _comb1_body,
        grid=(G,),
        in_specs=[_acc_spec(80), _row_spec(80), _row_spec(16),
                  _full_spec((1, 16)), _full_spec((1, 64)),
                  _full_spec((1, 64)), _full_spec((1, 64)),
                  _full_spec((64, 40)),
                  _full_spec((40, 1)), _full_spec((40, 1))],
        out_specs=[_row_spec(64), _row_spec(16), _full_spec((1, 16))],
        out_shape=(jax.ShapeDtypeStruct((N, 64), jnp.float32),
                   jax.ShapeDtypeStruct((N, 16), jnp.float32),
                   jax.ShapeDtypeStruct((1, 16), jnp.float32)),
    )(acc1, ha1, adst_t, bnd16, bias1.reshape(1, 64), gamma1.reshape(1, 64),
      beta1.reshape(1, 64), W2, as2m, ad2m)

    acc2 = _sc_edge(64, 48, 3, 40, True)(src, dst, ha2, adst2_t,
                                         bnd2.reshape(16))

    out = pl.pallas_call(
        _comb2_body,
        grid=(G,),
        in_specs=[_acc_spec(48), _row_spec(64), _row_spec(16),
                  _full_spec((1, 16)), _full_spec((40,)), _full_spec((40,)),
                  _full_spec((40,))],
        out_specs=_row_spec(40),
        out_shape=jax.ShapeDtypeStruct((N, 40), jnp.float32),
    )(acc2, ha2, adst2_t, bnd2, bias2, gamma2, beta2)
    return out


# speedup vs baseline: 1.0615x; 1.0615x over previous
import functools

import jax
import jax.numpy as jnp
import numpy as np
from jax import lax
from jax.experimental import pallas as pl
from jax.experimental.pallas import tpu as pltpu
from jax.experimental.pallas import tpu_sc as plsc

N = 10000
E = 320000
NC = 2
NS = 16
NW = NC * NS
EPW = E // NW
CH = 100
NCHUNK = EPW // CH
RPT = N // NS
ZR = 125


def _leaky(x):
    return jnp.where(x >= 0, x, 0.2 * x)


def _make_sc_edge(hw, rw, nb, ao, l2):
    mesh = plsc.VectorSubcoreMesh(core_axis_name="c", subcore_axis_name="s",
                                  num_cores=NC, num_subcores=NS)

    @functools.partial(
        pl.kernel,
        out_type=jax.ShapeDtypeStruct((NC, N, rw), jnp.float32),
        mesh=mesh,
        compiler_params=pltpu.CompilerParams(use_tc_tiling_on_sc=False),
        scratch_types=[
            pltpu.VMEM((NCHUNK, CH), jnp.int32),
            pltpu.VMEM((NCHUNK, CH), jnp.int32),
            pltpu.VMEM((2, CH, hw), jnp.float32),
            pltpu.VMEM((2, CH, 16), jnp.float32),
            pltpu.VMEM((2, CH, rw), jnp.float32),
            pltpu.VMEM((16,), jnp.float32),
            pltpu.VMEM((ZR, rw), jnp.float32),
            pltpu.VMEM_SHARED((N, rw), jnp.float32),
            pltpu.SemaphoreType.DMA,
            pltpu.SemaphoreType.DMA,
            pltpu.SemaphoreType.DMA,
            pltpu.SemaphoreType.DMA,
        ],
    )
    def sc_kernel(src_hbm, dst_hbm, ha_hbm, adst_hbm, bound_hbm,
                  acc_hbm, src_v, dst_v, ha_rows, ad_rows, res,
                  bnd_v, zb, acc_sh, sem0, sem1, ssem0, ssem1):
        c = lax.axis_index("c")
        s = lax.axis_index("s")
        wid = s * NC + c
        sems = (sem0, sem1)
        ssems = (ssem0, ssem1)

        def zrow(r, _):
            for k in range(rw // 16):
                zb[r, pl.ds(16 * k, 16)] = jnp.zeros((16,), jnp.float32)
            return 0
        lax.fori_loop(0, ZR, zrow, 0)
        for j in range(RPT // ZR):
            pltpu.sync_copy(zb, acc_sh.at[pl.ds(s * RPT + j * ZR, ZR)])
        plsc.subcore_barrier()

        pltpu.sync_copy(bound_hbm, bnd_v)
        pltpu.sync_copy(src_hbm.at[pl.ds(wid * NCHUNK, NCHUNK)], src_v)
        pltpu.sync_copy(dst_hbm.at[pl.ds(wid * NCHUNK, NCHUNK)], dst_v)
        bnd = bnd_v[...]
        lane = lax.iota(jnp.int32, 16)

        def gathers(i, b):
            return (pltpu.make_async_copy(ha_hbm.at[src_v.at[i]],
                                          ha_rows.at[b], sems[b]),
                    pltpu.make_async_copy(adst_hbm.at[dst_v.at[i]],
                                          ad_rows.at[b], sems[b]))

        def scatter_desc(i, b):
            return pltpu.make_async_copy(res.at[b], acc_sh.at[dst_v.at[i]],
                                         ssems[b])

        def compute(i, b, first=False):
            if not first:
                scatter_desc(i, b).wait()

            @plsc.parallel_loop(0, CH, step=1, unroll=8)
            def _edge(e):
                ex = jnp.exp(_leaky(ha_rows[b, e, pl.ds(ao, 16)]
                                    + ad_rows[b, e, :]) - bnd)
                for k in range(nb):
                    rb = ha_rows[b, e, pl.ds(16 * k, 16)] * ex
                    if l2 and k == nb - 1:
                        rb = jnp.where(lane < 8, rb,
                                       jnp.where(lane == 8, ex,
                                                 jnp.zeros((16,), jnp.float32)))
                    res[b, e, pl.ds(16 * k, 16)] = rb
                if not l2:
                    res[b, e, pl.ds(16 * nb, 16)] = ex
            pltpu.async_copy(res.at[b], acc_sh.at[dst_v.at[i]], ssems[b],
                             add=True)

        for d in gathers(0, 0):
            d.start()

        def pair(i0, first=False):
            d1 = gathers(i0 + 1, 1)
            for d in d1:
                d.start()
            for d in gathers(i0, 0):
                d.wait()
            compute(i0, 0, first=first)
            for d in gathers(i0 + 2, 0):
                d.start()
            for d in d1:
                d.wait()
            compute(i0 + 1, 1, first=first)

        def dbl(g, _):
            pair(2 * g)
            return 0
        pair(0, first=True)
        if NCHUNK % 2:
            lax.fori_loop(1, NCHUNK // 2, dbl, 0)
            for d in gathers(NCHUNK - 1, 0):
                d.wait()
            compute(NCHUNK - 1, 0)
        else:
            lax.fori_loop(1, NCHUNK // 2 - 1, dbl, 0)
            dlast = gathers(NCHUNK - 1, 1)
            for d in dlast:
                d.start()
            for d in gathers(NCHUNK - 2, 0):
                d.wait()
            compute(NCHUNK - 2, 0)
            for d in dlast:
                d.wait()
            compute(NCHUNK - 1, 1)
        scatter_desc(NCHUNK - 2, 0).wait()
        scatter_desc(NCHUNK - 1, 1).wait()
        plsc.subcore_barrier()

        pltpu.sync_copy(acc_sh.at[pl.ds(s * RPT, RPT)],
                        acc_hbm.at[c, pl.ds(s * RPT, RPT)])

    return sc_kernel


_sc_cache = {}


def _sc_edge(hw, rw, nb, ao, l2):
    key = (hw, rw, nb, ao, l2)
    if key not in _sc_cache:
        _sc_cache[key] = _make_sc_edge(hw, rw, nb, ao, l2)
    return _sc_cache[key]


BLK = 2000
G = N // BLK


def _perm_mat(transpose=False):
    i = lax.broadcasted_iota(jnp.int32, (64, 64), 1 if transpose else 0)
    k = lax.broadcasted_iota(jnp.int32, (64, 64), 0 if transpose else 1)
    return (i == (k % 8) * 8 + k // 8).astype(jnp.float32)


def _sum_mat():
    j = lax.broadcasted_iota(jnp.int32, (64, 8), 0)
    h = lax.broadcasted_iota(jnp.int32, (64, 8), 1)
    return (j // 8 == h).astype(jnp.float32)


def _prep1_body(x_ref, w1_ref, as1_ref, ad1_ref, ha_ref, adst_ref, bnd_ref):
    i = pl.program_id(0)
    hm = jnp.dot(x_ref[...], w1_ref[...], preferred_element_type=jnp.float32)
    sm = _sum_mat()
    asrc = jnp.dot(hm * as1_ref[...], sm, preferred_element_type=jnp.float32)
    adst = jnp.dot(hm * ad1_ref[...], sm, preferred_element_type=jnp.float32)
    h = jnp.dot(hm, _perm_mat(), preferred_element_type=jnp.float32)
    ha_ref[...] = jnp.concatenate([h, asrc, asrc], axis=1)
    adst_ref[...] = jnp.concatenate([adst, adst], axis=1)
    cur = jnp.concatenate([jnp.max(asrc, axis=0),
                           jnp.max(adst, axis=0)]).reshape(1, 16)
    acc = jnp.where(i == 0, cur, jnp.maximum(bnd_ref[...], cur))
    bnd_ref[...] = acc

    @pl.when(i == G - 1)
    def _():
        b = jnp.maximum(bnd_ref[0, :8] + bnd_ref[0, 8:], 0.0)
        bnd_ref[...] = jnp.concatenate([b, b]).reshape(1, 16)


def _comb1_body(acc_ref, ha_ref, adst_ref, bnd_ref, b1_ref, g1_ref,
                be1_ref, w2_ref, as2_ref, ad2_ref,
                ha2_ref, adst2_ref, bnd2_ref):
    asrc = ha_ref[:, 64:72]
    adst = adst_ref[:, :8]
    exs = jnp.exp(_leaky(asrc + adst) - bnd_ref[0, :8][None, :])
    den = acc_ref[0, :, 64:72] + acc_ref[1, :, 64:72] + exs
    exs_t = jnp.concatenate([exs] * 8, axis=1)
    den_t = jnp.concatenate([den] * 8, axis=1)
    num = acc_ref[0, :, :64] + acc_ref[1, :, :64] + exs_t * ha_ref[:, :64]
    pm = _perm_mat()
    z = num / den_t + jnp.dot(b1_ref[...], pm)
    mu = jnp.mean(z, axis=-1, keepdims=True)
    var = jnp.mean((z - mu) ** 2, axis=-1, keepdims=True)
    z = (z - mu) * lax.rsqrt(var + 1e-5) * jnp.dot(g1_ref[...], pm) \
        + jnp.dot(be1_ref[...], pm)
    z = jnp.where(z > 0, z, jnp.exp(jnp.minimum(z, 0.0)) - 1.0)
    w2p = jnp.dot(_perm_mat(transpose=True), w2_ref[...],
                  preferred_element_type=jnp.float32)
    h2 = jnp.dot(z, w2p, preferred_element_type=jnp.float32)
    a2s = jnp.dot(h2, as2_ref[...], preferred_element_type=jnp.float32)
    a2d = jnp.dot(h2, ad2_ref[...], preferred_element_type=jnp.float32)
    ha2_ref[...] = jnp.concatenate(
        [h2] + [a2s] * 16 + [jnp.zeros((h2.shape[0], 8), jnp.float32)], axis=1)
    adst2_ref[...] = jnp.concatenate([a2d] * 16, axis=1)
    i = pl.program_id(0)
    cur = jnp.concatenate(
        [jnp.max(a2s).reshape(1), jnp.max(a2d).reshape(1),
         jnp.full((14,), -3.0e38, jnp.float32)]).reshape(1, 16)
    acc = jnp.where(i == 0, cur, jnp.maximum(bnd2_ref[...], cur))
    bnd2_ref[...] = acc

    @pl.when(i == G - 1)
    def _():
        b2 = jnp.maximum(bnd2_ref[0, 0] + bnd2_ref[0, 1], 0.0)
        bnd2_ref[...] = jnp.full((1, 16), b2, jnp.float32)


def _comb2_body(acc_ref, ha2_ref, adst2_ref, bnd2_ref, b2_ref,
                g2_ref, be2_ref, out_ref):
    ex2 = jnp.exp(_leaky(ha2_ref[:, 40:41] + adst2_ref[:, :1]) - bnd2_ref[0, 0])
    den = acc_ref[0, :, 40:41] + acc_ref[1, :, 40:41] + ex2
    num = acc_ref[0, :, :40] + acc_ref[1, :, :40] + ex2 * ha2_ref[:, :40]
    z = num / den + b2_ref[...][None, :]
    mu = jnp.mean(z, axis=-1, keepdims=True)
    var = jnp.mean((z - mu) ** 2, axis=-1, keepdims=True)
    out_ref[...] = (z - mu) * lax.rsqrt(var + 1e-5) * g2_ref[...][None, :] \
        + be2_ref[...][None, :]


def _row_spec(w):
    return pl.BlockSpec((BLK, w), lambda i: (i, 0))


def _full_spec(shape):
    nd = len(shape)
    return pl.BlockSpec(shape, lambda i: (0,) * nd)


def _acc_spec(w):
    return pl.BlockSpec((NC, BLK, w), lambda i: (0, i, 0))


def kernel(x, edge_index, W1, att_src1, att_dst1, bias1, gamma1, beta1,
           W2, att_src2, att_dst2, bias2, gamma2, beta2):
    src = edge_index[0].reshape(E // CH, CH)
    dst = edge_index[1].reshape(E // CH, CH)
    as2m = att_src2.reshape(40, 1)
    ad2m = att_dst2.reshape(40, 1)

    ha1, adst_t, bnd16 = pl.pallas_call(
        _prep1_body,
        grid=(G,),
        in_specs=[_row_spec(128), _full_spec((128, 64)), _full_spec((1, 64)),
                  _full_spec((1, 64))],
        out_specs=[_row_spec(80), _row_spec(16), _full_spec((1, 16))],
        out_shape=(jax.ShapeDtypeStruct((N, 80), jnp.float32),
                   jax.ShapeDtypeStruct((N, 16), jnp.float32),
                   jax.ShapeDtypeStruct((1, 16), jnp.float32)),
    )(x, W1, att_src1.reshape(1, 64), att_dst1.reshape(1, 64))

    acc1 = _sc_edge(80, 80, 4, 64, False)(src, dst, ha1, adst_t,
                                          bnd16.reshape(16))

    ha2, adst2_t, bnd2 = pl.pallas_call(
        _comb1_body,
        grid=(G,),
        in_specs=[_acc_spec(80), _row_spec(80), _row_spec(16),
                  _full_spec((1, 16)), _full_spec((1, 64)),
                  _full_spec((1, 64)), _full_spec((1, 64)),
                  _full_spec((64, 40)),
                  _full_spec((40, 1)), _full_spec((40, 1))],
        out_specs=[_row_spec(64), _row_spec(16), _full_spec((1, 16))],
        out_shape=(jax.ShapeDtypeStruct((N, 64), jnp.float32),
                   jax.ShapeDtypeStruct((N, 16), jnp.float32),
                   jax.ShapeDtypeStruct((1, 16), jnp.float32)),
    )(acc1, ha1, adst_t, bnd16, bias1.reshape(1, 64), gamma1.reshape(1, 64),
      beta1.reshape(1, 64), W2, as2m, ad2m)

    acc2 = _sc_edge(64, 48, 3, 40, True)(src, dst, ha2, adst2_t,
                                         bnd2.reshape(16))

    out = pl.pallas_call(
        _comb2_body,
        grid=(G,),
        in_specs=[_acc_spec(48), _row_spec(64), _row_spec(16),
                  _full_spec((1, 16)), _full_spec((40,)), _full_spec((40,)),
                  _full_spec((40,))],
        out_specs=_row_spec(40),
        out_shape=jax.ShapeDtypeStruct((N, 40), jnp.float32),
    )(acc2, ha2, adst2_t, bnd2, bias2, gamma2, beta2)
    return out


# speedup vs baseline: 1.0941x; 1.0307x over previous
import functools

import jax
import jax.numpy as jnp
import numpy as np
from jax import lax
from jax.experimental import pallas as pl
from jax.experimental.pallas import tpu as pltpu
from jax.experimental.pallas import tpu_sc as plsc

N = 10000
E = 320000
NC = 2
NS = 16
NW = NC * NS
EPW = E // NW
CH = 125
NCHUNK = EPW // CH
RPT = N // NS
ZR = 125


def _leaky(x):
    return jnp.where(x >= 0, x, 0.2 * x)


def _make_sc_edge(hw, rw, nb, ao, l2):
    mesh = plsc.VectorSubcoreMesh(core_axis_name="c", subcore_axis_name="s",
                                  num_cores=NC, num_subcores=NS)

    @functools.partial(
        pl.kernel,
        out_type=jax.ShapeDtypeStruct((NC, N, rw), jnp.float32),
        mesh=mesh,
        compiler_params=pltpu.CompilerParams(use_tc_tiling_on_sc=False),
        scratch_types=[
            pltpu.VMEM((NCHUNK, CH), jnp.int32),
            pltpu.VMEM((NCHUNK, CH), jnp.int32),
            pltpu.VMEM((2, CH, hw), jnp.float32),
            pltpu.VMEM((2, CH, 16), jnp.float32),
            pltpu.VMEM((2, CH, rw), jnp.float32),
            pltpu.VMEM((16,), jnp.float32),
            pltpu.VMEM((ZR, rw), jnp.float32),
            pltpu.VMEM_SHARED((N, rw), jnp.float32),
            pltpu.SemaphoreType.DMA,
            pltpu.SemaphoreType.DMA,
            pltpu.SemaphoreType.DMA,
            pltpu.SemaphoreType.DMA,
        ],
    )
    def sc_kernel(src_hbm, dst_hbm, ha_hbm, adst_hbm, bound_hbm,
                  acc_hbm, src_v, dst_v, ha_rows, ad_rows, res,
                  bnd_v, zb, acc_sh, sem0, sem1, ssem0, ssem1):
        c = lax.axis_index("c")
        s = lax.axis_index("s")
        wid = s * NC + c
        sems = (sem0, sem1)
        ssems = (ssem0, ssem1)

        def zrow(r, _):
            for k in range(rw // 16):
                zb[r, pl.ds(16 * k, 16)] = jnp.zeros((16,), jnp.float32)
            return 0
        lax.fori_loop(0, ZR, zrow, 0)
        for j in range(RPT // ZR):
            pltpu.sync_copy(zb, acc_sh.at[pl.ds(s * RPT + j * ZR, ZR)])
        plsc.subcore_barrier()

        pltpu.sync_copy(bound_hbm, bnd_v)
        pltpu.sync_copy(src_hbm.at[pl.ds(wid * NCHUNK, NCHUNK)], src_v)
        pltpu.sync_copy(dst_hbm.at[pl.ds(wid * NCHUNK, NCHUNK)], dst_v)
        bnd = bnd_v[...]
        lane = lax.iota(jnp.int32, 16)

        def gathers(i, b):
            return (pltpu.make_async_copy(ha_hbm.at[src_v.at[i]],
                                          ha_rows.at[b], sems[b]),
                    pltpu.make_async_copy(adst_hbm.at[dst_v.at[i]],
                                          ad_rows.at[b], sems[b]))

        def scatter_desc(i, b):
            return pltpu.make_async_copy(res.at[b], acc_sh.at[dst_v.at[i]],
                                         ssems[b])

        def compute(i, b, first=False):
            if not first:
                scatter_desc(i, b).wait()

            @plsc.parallel_loop(0, CH, step=1, unroll=16)
            def _edge(e):
                ex = jnp.exp(_leaky(ha_rows[b, e, pl.ds(ao, 16)]
                                    + ad_rows[b, e, :]) - bnd)
                for k in range(nb):
                    rb = ha_rows[b, e, pl.ds(16 * k, 16)] * ex
                    if l2 and k == nb - 1:
                        rb = jnp.where(lane < 8, rb,
                                       jnp.where(lane == 8, ex,
                                                 jnp.zeros((16,), jnp.float32)))
                    res[b, e, pl.ds(16 * k, 16)] = rb
                if not l2:
                    res[b, e, pl.ds(16 * nb, 16)] = ex
            pltpu.async_copy(res.at[b], acc_sh.at[dst_v.at[i]], ssems[b],
                             add=True)

        for d in gathers(0, 0):
            d.start()

        def pair(i0, first=False):
            d1 = gathers(i0 + 1, 1)
            for d in d1:
                d.start()
            for d in gathers(i0, 0):
                d.wait()
            compute(i0, 0, first=first)
            for d in gathers(i0 + 2, 0):
                d.start()
            for d in d1:
                d.wait()
            compute(i0 + 1, 1, first=first)

        def dbl(g, _):
            pair(2 * g)
            return 0
        pair(0, first=True)
        if NCHUNK % 2:
            lax.fori_loop(1, NCHUNK // 2, dbl, 0)
            for d in gathers(NCHUNK - 1, 0):
                d.wait()
            compute(NCHUNK - 1, 0)
        else:
            lax.fori_loop(1, NCHUNK // 2 - 1, dbl, 0)
            dlast = gathers(NCHUNK - 1, 1)
            for d in dlast:
                d.start()
            for d in gathers(NCHUNK - 2, 0):
                d.wait()
            compute(NCHUNK - 2, 0)
            for d in dlast:
                d.wait()
            compute(NCHUNK - 1, 1)
        scatter_desc(NCHUNK - 2, 0).wait()
        scatter_desc(NCHUNK - 1, 1).wait()
        plsc.subcore_barrier()

        pltpu.sync_copy(acc_sh.at[pl.ds(s * RPT, RPT)],
                        acc_hbm.at[c, pl.ds(s * RPT, RPT)])

    return sc_kernel


_sc_cache = {}


def _sc_edge(hw, rw, nb, ao, l2):
    key = (hw, rw, nb, ao, l2)
    if key not in _sc_cache:
        _sc_cache[key] = _make_sc_edge(hw, rw, nb, ao, l2)
    return _sc_cache[key]


BLK = 2000
G = N // BLK


def _perm_mat(transpose=False):
    i = lax.broadcasted_iota(jnp.int32, (64, 64), 1 if transpose else 0)
    k = lax.broadcasted_iota(jnp.int32, (64, 64), 0 if transpose else 1)
    return (i == (k % 8) * 8 + k // 8).astype(jnp.float32)


def _sum_mat():
    j = lax.broadcasted_iota(jnp.int32, (64, 8), 0)
    h = lax.broadcasted_iota(jnp.int32, (64, 8), 1)
    return (j // 8 == h).astype(jnp.float32)


def _prep1_body(x_ref, w1_ref, as1_ref, ad1_ref, ha_ref, adst_ref, bnd_ref):
    i = pl.program_id(0)
    hm = jnp.dot(x_ref[...], w1_ref[...], preferred_element_type=jnp.float32)
    sm = _sum_mat()
    asrc = jnp.dot(hm * as1_ref[...], sm, preferred_element_type=jnp.float32)
    adst = jnp.dot(hm * ad1_ref[...], sm, preferred_element_type=jnp.float32)
    h = jnp.dot(hm, _perm_mat(), preferred_element_type=jnp.float32)
    ha_ref[...] = jnp.concatenate([h, asrc, asrc], axis=1)
    adst_ref[...] = jnp.concatenate([adst, adst], axis=1)
    cur = jnp.concatenate([jnp.max(asrc, axis=0),
                           jnp.max(adst, axis=0)]).reshape(1, 16)
    acc = jnp.where(i == 0, cur, jnp.maximum(bnd_ref[...], cur))
    bnd_ref[...] = acc

    @pl.when(i == G - 1)
    def _():
        b = jnp.maximum(bnd_ref[0, :8] + bnd_ref[0, 8:], 0.0)
        bnd_ref[...] = jnp.concatenate([b, b]).reshape(1, 16)


def _comb1_body(acc_ref, ha_ref, adst_ref, bnd_ref, b1_ref, g1_ref,
                be1_ref, w2_ref, as2_ref, ad2_ref,
                ha2_ref, adst2_ref, bnd2_ref):
    asrc = ha_ref[:, 64:72]
    adst = adst_ref[:, :8]
    exs = jnp.exp(_leaky(asrc + adst) - bnd_ref[0, :8][None, :])
    den = acc_ref[0, :, 64:72] + acc_ref[1, :, 64:72] + exs
    exs_t = jnp.concatenate([exs] * 8, axis=1)
    den_t = jnp.concatenate([den] * 8, axis=1)
    num = acc_ref[0, :, :64] + acc_ref[1, :, :64] + exs_t * ha_ref[:, :64]
    pm = _perm_mat()
    z = num / den_t + jnp.dot(b1_ref[...], pm)
    mu = jnp.mean(z, axis=-1, keepdims=True)
    var = jnp.mean((z - mu) ** 2, axis=-1, keepdims=True)
    z = (z - mu) * lax.rsqrt(var + 1e-5) * jnp.dot(g1_ref[...], pm) \
        + jnp.dot(be1_ref[...], pm)
    z = jnp.where(z > 0, z, jnp.exp(jnp.minimum(z, 0.0)) - 1.0)
    w2p = jnp.dot(_perm_mat(transpose=True), w2_ref[...],
                  preferred_element_type=jnp.float32)
    h2 = jnp.dot(z, w2p, preferred_element_type=jnp.float32)
    a2s = jnp.dot(h2, as2_ref[...], preferred_element_type=jnp.float32)
    a2d = jnp.dot(h2, ad2_ref[...], preferred_element_type=jnp.float32)
    ha2_ref[...] = jnp.concatenate(
        [h2] + [a2s] * 16 + [jnp.zeros((h2.shape[0], 8), jnp.float32)], axis=1)
    adst2_ref[...] = jnp.concatenate([a2d] * 16, axis=1)
    i = pl.program_id(0)
    cur = jnp.concatenate(
        [jnp.max(a2s).reshape(1), jnp.max(a2d).reshape(1),
         jnp.full((14,), -3.0e38, jnp.float32)]).reshape(1, 16)
    acc = jnp.where(i == 0, cur, jnp.maximum(bnd2_ref[...], cur))
    bnd2_ref[...] = acc

    @pl.when(i == G - 1)
    def _():
        b2 = jnp.maximum(bnd2_ref[0, 0] + bnd2_ref[0, 1], 0.0)
        bnd2_ref[...] = jnp.full((1, 16), b2, jnp.float32)


def _comb2_body(acc_ref, ha2_ref, adst2_ref, bnd2_ref, b2_ref,
                g2_ref, be2_ref, out_ref):
    ex2 = jnp.exp(_leaky(ha2_ref[:, 40:41] + adst2_ref[:, :1]) - bnd2_ref[0, 0])
    den = acc_ref[0, :, 40:41] + acc_ref[1, :, 40:41] + ex2
    num = acc_ref[0, :, :40] + acc_ref[1, :, :40] + ex2 * ha2_ref[:, :40]
    z = num / den + b2_ref[...][None, :]
    mu = jnp.mean(z, axis=-1, keepdims=True)
    var = jnp.mean((z - mu) ** 2, axis=-1, keepdims=True)
    out_ref[...] = (z - mu) * lax.rsqrt(var + 1e-5) * g2_ref[...][None, :] \
        + be2_ref[...][None, :]


def _row_spec(w):
    return pl.BlockSpec((BLK, w), lambda i: (i, 0))


def _full_spec(shape):
    nd = len(shape)
    return pl.BlockSpec(shape, lambda i: (0,) * nd)


def _acc_spec(w):
    return pl.BlockSpec((NC, BLK, w), lambda i: (0, i, 0))


def kernel(x, edge_index, W1, att_src1, att_dst1, bias1, gamma1, beta1,
           W2, att_src2, att_dst2, bias2, gamma2, beta2):
    src = edge_index[0].reshape(E // CH, CH)
    dst = edge_index[1].reshape(E // CH, CH)
    as2m = att_src2.reshape(40, 1)
    ad2m = att_dst2.reshape(40, 1)

    ha1, adst_t, bnd16 = pl.pallas_call(
        _prep1_body,
        grid=(G,),
        in_specs=[_row_spec(128), _full_spec((128, 64)), _full_spec((1, 64)),
                  _full_spec((1, 64))],
        out_specs=[_row_spec(80), _row_spec(16), _full_spec((1, 16))],
        out_shape=(jax.ShapeDtypeStruct((N, 80), jnp.float32),
                   jax.ShapeDtypeStruct((N, 16), jnp.float32),
                   jax.ShapeDtypeStruct((1, 16), jnp.float32)),
    )(x, W1, att_src1.reshape(1, 64), att_dst1.reshape(1, 64))

    acc1 = _sc_edge(80, 80, 4, 64, False)(src, dst, ha1, adst_t,
                                          bnd16.reshape(16))

    ha2, adst2_t, bnd2 = pl.pallas_call(
        _comb1_body,
        grid=(G,),
        in_specs=[_acc_spec(80), _row_spec(80), _row_spec(16),
                  _full_spec((1, 16)), _full_spec((1, 64)),
                  _full_spec((1, 64)), _full_spec((1, 64)),
                  _full_spec((64, 40)),
                  _full_spec((40, 1)), _full_spec((40, 1))],
        out_specs=[_row_spec(64), _row_spec(16), _full_spec((1, 16))],
        out_shape=(jax.ShapeDtypeStruct((N, 64), jnp.float32),
                   jax.ShapeDtypeStruct((N, 16), jnp.float32),
                   jax.ShapeDtypeStruct((1, 16), jnp.float32)),
    )(acc1, ha1, adst_t, bnd16, bias1.reshape(1, 64), gamma1.reshape(1, 64),
      beta1.reshape(1, 64), W2, as2m, ad2m)

    acc2 = _sc_edge(64, 48, 3, 40, True)(src, dst, ha2, adst2_t,
                                         bnd2.reshape(16))

    out = pl.pallas_call(
        _comb2_body,
        grid=(G,),
        in_specs=[_acc_spec(48), _row_spec(64), _row_spec(16),
                  _full_spec((1, 16)), _full_spec((40,)), _full_spec((40,)),
                  _full_spec((40,))],
        out_specs=_row_spec(40),
        out_shape=jax.ShapeDtypeStruct((N, 40), jnp.float32),
    )(acc2, ha2, adst2_t, bnd2, bias2, gamma2, beta2)
    return out
